# baseline (speedup 1.0000x reference)
import functools

import jax
import jax.numpy as jnp
from jax import lax
from jax.experimental import pallas as pl
from jax.experimental.pallas import tpu as pltpu
from jax.experimental.pallas import tpu_sc as plsc

NN = 10000
NP = 10240
NE = 160000
EP = 163840
H = 256
NG = 64
RB = 512
NRB = NP // RB

XW = 16
EPW = EP // 32
DROWS = EPW // 128
XCH = 8
XNCH = DROWS // XCH
NHP = NP
XACC = NP
XRPT = NP // 16
XZR = NP // 16

EPT = EP // 16
MROWS = EPT // 128
CH = 4
NCH = MROWS // CH
RPT = NP // 16
HQ = 128
NPH = 1

_f32 = jnp.float32
_i32 = jnp.int32
_bf16 = jnp.bfloat16


@functools.cache
def _sc_mesh():
    return plsc.VectorSubcoreMesh(core_axis_name="c", subcore_axis_name="s",
                                  num_cores=2, num_subcores=16)


_SC_PARAMS = pltpu.CompilerParams(use_tc_tiling_on_sc=False)


@functools.cache
def _deg_kernel_fn():
    return pl.kernel(
        _deg_body,
        out_type=jax.ShapeDtypeStruct((2, NP, XW), _f32),
        mesh=_sc_mesh(),
        scratch_types=[
            pltpu.VMEM((DROWS, 128), _i32),
            pltpu.VMEM((128, XW), _f32),
            pltpu.VMEM_SHARED((XACC, XW), _f32),
        ],
        compiler_params=_SC_PARAMS,
    )


def _deg_body(dst_hbm, ones_hbm, zeros_hbm, out_hbm, idx_v, ones_v, acc_sh):
    c = lax.axis_index("c")
    s = lax.axis_index("s")
    pltpu.sync_copy(dst_hbm.at[c, s], idx_v)
    pltpu.sync_copy(ones_hbm, ones_v)
    pltpu.sync_copy(zeros_hbm, acc_sh.at[pl.ds(s * XZR, XZR)])
    plsc.subcore_barrier()
    for r in range(DROWS):
        pltpu.sync_copy(ones_v, acc_sh.at[idx_v.at[r]], add=True)
    plsc.subcore_barrier()
    pltpu.sync_copy(acc_sh.at[pl.ds(s * XRPT, XRPT)],
                    out_hbm.at[c, pl.ds(s * XRPT, XRPT)])


def _edge_pipeline(src_ref, dst_ref, gather_hbm, bufs, acc_sh, gsem, ssem,
                   nchunks, ch, idx_bufs=None, shift=0, q=None):
    def gen_idx(i, b):
        if idx_bufs is None:
            return
        for j in range(ch):
            for k in range(8):
                sl = src_ref[i * ch + j, pl.ds(k * 16, 16)]
                idx_bufs[b][j, pl.ds(k * 16, 16)] = (sl << shift) + q

    def fire_gathers(i, b):
        iref = src_ref if idx_bufs is None else idx_bufs[b]
        off = i * ch if idx_bufs is None else 0
        return [
            pltpu.async_copy(gather_hbm.at[iref.at[off + j]],
                             bufs[b].at[pl.ds(j * 128, 128)], gsem)
            for j in range(ch)
        ]

    def fire_scatters(i, b):
        return [
            pltpu.async_copy(bufs[b].at[pl.ds(j * 128, 128)],
                             acc_sh.at[dst_ref.at[i * ch + j]], ssem,
                             add=True)
            for j in range(ch)
        ]

    scat = [None, None]
    gen_idx(0, 0)
    gh = fire_gathers(0, 0)
    for i in range(nchunks):
        b = i % 2
        if i + 1 < nchunks:
            gen_idx(i + 1, (i + 1) % 2)
        for h_ in gh:
            h_.wait()
        if i + 1 < nchunks:
            nb = (i + 1) % 2
            if scat[nb] is not None:
                for h_ in scat[nb]:
                    h_.wait()
            gh = fire_gathers(i + 1, nb)
        scat[b] = fire_scatters(i, b)
    for sl in scat:
        if sl is not None:
            for h_ in sl:
                h_.wait()


@functools.cache
def _msgx_kernel_fn():
    return pl.kernel(
        _msgx_body,
        out_type=jax.ShapeDtypeStruct((2, NP, XW), _f32),
        mesh=_sc_mesh(),
        scratch_types=[
            pltpu.VMEM((DROWS, 128), _i32),
            pltpu.VMEM((DROWS, 128), _i32),
            pltpu.VMEM((XCH * 128, XW), _f32),
            pltpu.VMEM((XCH * 128, XW), _f32),
            pltpu.VMEM_SHARED((XACC, XW), _f32),
            pltpu.SemaphoreType.DMA,
            pltpu.SemaphoreType.DMA,
        ],
        compiler_params=_SC_PARAMS,
    )


def _msgx_body(ux_hbm, src_hbm, dst_hbm, zeros_hbm, out_hbm,
               src_v, dst_v, rows0, rows1, acc_sh, gsem, ssem):
    c = lax.axis_index("c")
    s = lax.axis_index("s")
    pltpu.sync_copy(src_hbm.at[c, s], src_v)
    pltpu.sync_copy(dst_hbm.at[c, s], dst_v)
    pltpu.sync_copy(zeros_hbm, acc_sh.at[pl.ds(s * XZR, XZR)])
    plsc.subcore_barrier()
    _edge_pipeline(src_v, dst_v, ux_hbm, (rows0, rows1), acc_sh,
                   gsem, ssem, XNCH, XCH)
    plsc.subcore_barrier()
    pltpu.sync_copy(acc_sh.at[pl.ds(s * XRPT, XRPT)],
                    out_hbm.at[c, pl.ds(s * XRPT, XRPT)])


@functools.cache
def _msg_kernel_fn():
    return pl.kernel(
        _msg_body,
        out_type=jax.ShapeDtypeStruct((2, NP, HQ), _bf16),
        mesh=_sc_mesh(),
        scratch_types=[
            pltpu.VMEM((MROWS, 128), _i32),
            pltpu.VMEM((MROWS, 128), _i32),
            pltpu.VMEM((CH, 128), _i32),
            pltpu.VMEM((CH, 128), _i32),
            pltpu.VMEM((CH * 128, HQ), _bf16),
            pltpu.VMEM((CH * 128, HQ), _bf16),
            pltpu.VMEM_SHARED((NP, HQ), _bf16),
            pltpu.SemaphoreType.DMA,
            pltpu.SemaphoreType.DMA,
        ],
        compiler_params=_SC_PARAMS,
    )


def _msg_body(u2_hbm, src_hbm, dst_hbm, zeros_hbm, out_hbm,
              src_v, dst_v, idx0, idx1, rows0, rows1, acc_sh, gsem, ssem):
    c = lax.axis_index("c")
    s = lax.axis_index("s")
    pltpu.sync_copy(src_hbm.at[s], src_v)
    pltpu.sync_copy(dst_hbm.at[s], dst_v)
    pltpu.sync_copy(zeros_hbm, acc_sh.at[pl.ds(s * RPT, RPT)])
    plsc.subcore_barrier()
    _edge_pipeline(src_v, dst_v, u2_hbm, (rows0, rows1), acc_sh,
                   gsem, ssem, NCH, CH, idx_bufs=(idx0, idx1), shift=1,
                   q=c)
    plsc.subcore_barrier()
    pltpu.sync_copy(acc_sh.at[pl.ds(s * RPT, RPT)],
                    out_hbm.at[c, pl.ds(s * RPT, RPT)])


def _dinv_block(degp_ref):
    deg = degp_ref[0, :, 0] + degp_ref[1, :, 0] + 1.0
    return lax.rsqrt(jnp.maximum(deg, 1.0))


def _ux_body(x_ref, degp_ref, ux_ref):
    dinv = _dinv_block(degp_ref)
    ux_ref[...] = x_ref[...] * dinv[:, None]


def _l1_body(sx_ref, ux_ref, degp_ref, b1_ref, w1_ref, w2_ref, u2_ref,
             u2b_ref):
    dinv = _dinv_block(degp_ref)
    t = dinv[:, None] * (sx_ref[0] + sx_ref[1] + ux_ref[...])
    h1 = jnp.maximum(
        jnp.dot(t, w1_ref[...], preferred_element_type=_f32) + b1_ref[...],
        0.0)
    u2 = jnp.dot(h1, w2_ref[...],
                 preferred_element_type=_f32) * dinv[:, None]
    u2_ref[...] = u2
    u2b_ref[...] = u2.astype(_bf16).reshape(2 * RB, HQ)


def _mid_body(s_ref, u_ref, degp_ref, b_ref, w_ref, un_ref, unb_ref):
    dinv = _dinv_block(degp_ref)
    sc = jnp.concatenate([s_ref[0], s_ref[1]], axis=-1).astype(_f32)
    h = jnp.maximum(dinv[:, None] * (sc + u_ref[...]) + b_ref[...], 0.0)
    un = jnp.dot(h, w_ref[...],
                 preferred_element_type=_f32) * dinv[:, None]
    un_ref[...] = un
    unb_ref[...] = un.astype(_bf16).reshape(2 * RB, HQ)


def _final_body(s_ref, u_ref, degp_ref, b_ref, batch_ref,
                lw1_ref, lb1_ref, lw2_ref, lb2_ref, lw3_ref, lb3_ref,
                logp_ref, feat_ref, pool_acc, cnt_acc):
    i = pl.program_id(0)

    @pl.when(i == 0)
    def _():
        pool_acc[...] = jnp.zeros_like(pool_acc)
        cnt_acc[...] = jnp.zeros_like(cnt_acc)

    dinv = _dinv_block(degp_ref)
    sc = jnp.concatenate([s_ref[0], s_ref[1]], axis=-1).astype(_f32)
    h = jnp.maximum(dinv[:, None] * (sc + u_ref[...]) + b_ref[...], 0.0)
    b_row = batch_ref[0, 0, :]
    gi = lax.broadcasted_iota(_i32, (NG, RB), 0)
    onehot = (b_row[None, :] == gi).astype(_f32)
    pool_acc[...] += jnp.dot(onehot, h, preferred_element_type=_f32)
    cnt_acc[...] += jnp.sum(onehot, axis=1)[:, None]

    @pl.when(i == pl.num_programs(0) - 1)
    def _():
        counts = jnp.maximum(cnt_acc[:, 0:1], 1.0)
        g = pool_acc[...] / counts
        g = jnp.maximum(
            jnp.dot(g, lw1_ref[...], preferred_element_type=_f32)
            + lb1_ref[...], 0.0)
        feat = jnp.dot(g, lw2_ref[...],
                       preferred_element_type=_f32) + lb2_ref[...]
        logits = jnp.dot(feat, lw3_ref[...],
                         preferred_element_type=_f32) + lb3_ref[...]
        col = lax.broadcasted_iota(_i32, (NG, 128), 1)
        ml = jnp.where(col < 30, logits, jnp.full_like(logits, -1e30))
        mx = jnp.max(ml, axis=-1, keepdims=True)
        lse = mx + jnp.log(jnp.sum(jnp.exp(ml - mx), axis=-1, keepdims=True))
        logp_ref[...] = ml - lse
        feat_ref[...] = feat


def _row_spec(width):
    return pl.BlockSpec((RB, width), lambda i: (i, 0))


_S_SPEC = pl.BlockSpec((2, RB, HQ), lambda i: (0, i, 0))
_SX_SPEC = pl.BlockSpec((2, RB, XW), lambda i: (0, i, 0))
_DEG_SPEC = pl.BlockSpec((2, RB, XW), lambda i: (0, i, 0))


def _full_spec(shape):
    nd = len(shape)
    return pl.BlockSpec(shape, lambda i, _n=nd: (0,) * _n)


def kernel(x, edge_index, batch, W1, b1, W2, b2, W3, b3,
           lw1, lb1, lw2, lb2, lw3, lb3):
    src = edge_index[0].astype(_i32)
    dst = edge_index[1].astype(_i32)
    pad_e = EP - NE
    src_p = jnp.concatenate([src, jnp.zeros((pad_e,), _i32)])
    dst_p = jnp.concatenate([dst, jnp.full((pad_e,), NN, _i32)])
    src_w = src_p.reshape(2, 16, DROWS, 128)
    dst_w = dst_p.reshape(2, 16, DROWS, 128)
    src_msg = src_p.reshape(16, MROWS, 128)
    dst_msg = dst_p.reshape(16, MROWS, 128)
    batch3 = jnp.concatenate(
        [batch.astype(_i32),
         jnp.full((NP - NN,), NG, _i32)]).reshape(NRB, 1, RB)
    x16 = jnp.zeros((NP, XW), _f32).at[:NN, :7].set(x)
    W1p = jnp.zeros((XW, H), _f32).at[:7].set(W1)
    ones_deg = jnp.ones((128, XW), _f32)
    zeros_x = jnp.zeros((XZR, XW), _f32)
    zeros_msg = jnp.zeros((RPT, HQ), _bf16)
    lw3p = jnp.zeros((H, 128), _f32).at[:, :30].set(lw3)
    lb3p = jnp.zeros((1, 128), _f32).at[0, :30].set(lb3)
    b1r, b2r, b3r = (b.reshape(1, H) for b in (b1, b2, b3))
    lb1r, lb2r = lb1.reshape(1, H), lb2.reshape(1, 256)

    degp = _deg_kernel_fn()(dst_w, ones_deg, zeros_x)

    ux = pl.pallas_call(
        _ux_body,
        grid=(NRB,),
        in_specs=[_row_spec(XW), _DEG_SPEC],
        out_specs=_row_spec(XW),
        out_shape=jax.ShapeDtypeStruct((NP, XW), _f32),
    )(x16, degp)

    sx = _msgx_kernel_fn()(ux, src_w, dst_w, zeros_x)

    u2, u2b = pl.pallas_call(
        _l1_body,
        grid=(NRB,),
        in_specs=[_SX_SPEC, _row_spec(XW), _DEG_SPEC,
                  _full_spec((1, H)), _full_spec((XW, H)),
                  _full_spec((H, H))],
        out_specs=[_row_spec(H),
                   pl.BlockSpec((2 * RB, HQ), lambda i: (i, 0))],
        out_shape=[jax.ShapeDtypeStruct((NP, H), _f32),
                   jax.ShapeDtypeStruct((2 * NP, HQ), _bf16)],
    )(sx, ux, degp, b1r, W1p, W2)

    def msg(u2v):
        return _msg_kernel_fn()(u2v, src_msg, dst_msg, zeros_msg)

    s2 = msg(u2b)

    u3, u3b = pl.pallas_call(
        _mid_body,
        grid=(NRB,),
        in_specs=[_S_SPEC, _row_spec(H), _DEG_SPEC,
                  _full_spec((1, H)), _full_spec((H, H))],
        out_specs=[_row_spec(H),
                   pl.BlockSpec((2 * RB, HQ), lambda i: (i, 0))],
        out_shape=[jax.ShapeDtypeStruct((NP, H), _f32),
                   jax.ShapeDtypeStruct((2 * NP, HQ), _bf16)],
    )(s2, u2, degp, b2r, W3)

    s3 = msg(u3b)

    logp_pad, feat = pl.pallas_call(
        _final_body,
        grid=(NRB,),
        in_specs=[_S_SPEC, _row_spec(H), _DEG_SPEC, _full_spec((1, H)),
                  pl.BlockSpec((1, 1, RB), lambda i: (i, 0, 0)),
                  _full_spec((H, H)), _full_spec((1, H)),
                  _full_spec((H, 256)), _full_spec((1, 256)),
                  _full_spec((H, 128)), _full_spec((1, 128))],
        out_specs=[pl.BlockSpec((NG, 128), lambda i: (0, 0)),
                   pl.BlockSpec((NG, 256), lambda i: (0, 0))],
        out_shape=[jax.ShapeDtypeStruct((NG, 128), _f32),
                   jax.ShapeDtypeStruct((NG, 256), _f32)],
        scratch_shapes=[pltpu.VMEM((NG, 256), _f32),
                        pltpu.VMEM((NG, 128), _f32)],
    )(s3, u3, degp, b3r, batch3, lw1, lb1r, lw2, lb2r, lw3p, lb3p)

    return (logp_pad[:, :30], feat)


# speedup vs baseline: 1.0165x; 1.0165x over previous
import functools

import jax
import jax.numpy as jnp
from jax import lax
from jax.experimental import pallas as pl
from jax.experimental.pallas import tpu as pltpu
from jax.experimental.pallas import tpu_sc as plsc

NN = 10000
NP = 10240
NE = 160000
EP = 163840
H = 256
NG = 64
RB = 512
NRB = NP // RB

XW = 16
EPW = EP // 32
DROWS = EPW // 128
XCH = 8
XNCH = DROWS // XCH
NHP = NP
XACC = NP
XRPT = NP // 16
XZR = NP // 16

EPT = EP // 16
MROWS = EPT // 128
CH = 4
NCH = MROWS // CH
RPT = NP // 16
HQ = 128
NPH = 1

_f32 = jnp.float32
_i32 = jnp.int32
_bf16 = jnp.bfloat16


@functools.cache
def _sc_mesh():
    return plsc.VectorSubcoreMesh(core_axis_name="c", subcore_axis_name="s",
                                  num_cores=2, num_subcores=16)


_SC_PARAMS = pltpu.CompilerParams(use_tc_tiling_on_sc=False)


@functools.cache
def _deg_kernel_fn():
    return pl.kernel(
        _deg_body,
        out_type=jax.ShapeDtypeStruct((2, NP, XW), _f32),
        mesh=_sc_mesh(),
        scratch_types=[
            pltpu.VMEM((DROWS, 128), _i32),
            pltpu.VMEM((128, XW), _f32),
            pltpu.VMEM_SHARED((XACC, XW), _f32),
        ],
        compiler_params=_SC_PARAMS,
    )


def _deg_body(dst_hbm, ones_hbm, zeros_hbm, out_hbm, idx_v, ones_v, acc_sh):
    c = lax.axis_index("c")
    s = lax.axis_index("s")
    pltpu.sync_copy(dst_hbm.at[c, s], idx_v)
    pltpu.sync_copy(ones_hbm, ones_v)
    pltpu.sync_copy(zeros_hbm, acc_sh.at[pl.ds(s * XZR, XZR)])
    plsc.subcore_barrier()
    for r in range(DROWS):
        pltpu.sync_copy(ones_v, acc_sh.at[idx_v.at[r]], add=True)
    plsc.subcore_barrier()
    pltpu.sync_copy(acc_sh.at[pl.ds(s * XRPT, XRPT)],
                    out_hbm.at[c, pl.ds(s * XRPT, XRPT)])


def _edge_pipeline(src_ref, dst_ref, gather_hbm, bufs, acc_sh, gsem, ssem,
                   nchunks, ch, idx_bufs=None, shift=0, q=None):
    def gen_idx(i, b):
        if idx_bufs is None:
            return
        for j in range(ch):
            for k in range(8):
                sl = src_ref[i * ch + j, pl.ds(k * 16, 16)]
                idx_bufs[b][j, pl.ds(k * 16, 16)] = (sl << shift) + q

    def fire_gathers(i, b):
        iref = src_ref if idx_bufs is None else idx_bufs[b]
        off = i * ch if idx_bufs is None else 0
        return [
            pltpu.async_copy(gather_hbm.at[iref.at[off + j]],
                             bufs[b].at[pl.ds(j * 128, 128)], gsem)
            for j in range(ch)
        ]

    def fire_scatters(i, b):
        return [
            pltpu.async_copy(bufs[b].at[pl.ds(j * 128, 128)],
                             acc_sh.at[dst_ref.at[i * ch + j]], ssem,
                             add=True)
            for j in range(ch)
        ]

    scat = [None, None]
    gen_idx(0, 0)
    gh = fire_gathers(0, 0)
    for i in range(nchunks):
        b = i % 2
        if i + 1 < nchunks:
            gen_idx(i + 1, (i + 1) % 2)
        for h_ in gh:
            h_.wait()
        if i + 1 < nchunks:
            nb = (i + 1) % 2
            if scat[nb] is not None:
                for h_ in scat[nb]:
                    h_.wait()
            gh = fire_gathers(i + 1, nb)
        scat[b] = fire_scatters(i, b)
    for sl in scat:
        if sl is not None:
            for h_ in sl:
                h_.wait()


@functools.cache
def _msgx_kernel_fn():
    return pl.kernel(
        _msgx_body,
        out_type=jax.ShapeDtypeStruct((2, NP, XW), _f32),
        mesh=_sc_mesh(),
        scratch_types=[
            pltpu.VMEM((DROWS, 128), _i32),
            pltpu.VMEM((DROWS, 128), _i32),
            pltpu.VMEM((XCH * 128, XW), _f32),
            pltpu.VMEM((XCH * 128, XW), _f32),
            pltpu.VMEM_SHARED((XACC, XW), _f32),
            pltpu.SemaphoreType.DMA,
            pltpu.SemaphoreType.DMA,
        ],
        compiler_params=_SC_PARAMS,
    )


def _msgx_body(ux_hbm, src_hbm, dst_hbm, zeros_hbm, out_hbm,
               src_v, dst_v, rows0, rows1, acc_sh, gsem, ssem):
    c = lax.axis_index("c")
    s = lax.axis_index("s")
    pltpu.sync_copy(src_hbm.at[c, s], src_v)
    pltpu.sync_copy(dst_hbm.at[c, s], dst_v)
    pltpu.sync_copy(zeros_hbm, acc_sh.at[pl.ds(s * XZR, XZR)])
    plsc.subcore_barrier()
    _edge_pipeline(src_v, dst_v, ux_hbm, (rows0, rows1), acc_sh,
                   gsem, ssem, XNCH, XCH)
    plsc.subcore_barrier()
    pltpu.sync_copy(acc_sh.at[pl.ds(s * XRPT, XRPT)],
                    out_hbm.at[c, pl.ds(s * XRPT, XRPT)])


@functools.cache
def _msg_kernel_fn():
    return pl.kernel(
        _msg_body,
        out_type=jax.ShapeDtypeStruct((2, NP, HQ), _bf16),
        mesh=_sc_mesh(),
        scratch_types=[
            pltpu.VMEM((MROWS, 128), _i32),
            pltpu.VMEM((MROWS, 128), _i32),
            pltpu.VMEM((CH, 128), _i32),
            pltpu.VMEM((CH, 128), _i32),
            pltpu.VMEM((CH * 128, HQ), _bf16),
            pltpu.VMEM((CH * 128, HQ), _bf16),
            pltpu.VMEM_SHARED((NP, HQ), _bf16),
            pltpu.SemaphoreType.DMA,
            pltpu.SemaphoreType.DMA,
        ],
        compiler_params=_SC_PARAMS,
    )


def _msg_body(u2_hbm, src_hbm, dst_hbm, zeros_hbm, out_hbm,
              src_v, dst_v, idx0, idx1, rows0, rows1, acc_sh, gsem, ssem):
    c = lax.axis_index("c")
    s = lax.axis_index("s")
    pltpu.sync_copy(src_hbm.at[s], src_v)
    pltpu.sync_copy(dst_hbm.at[s], dst_v)
    pltpu.sync_copy(zeros_hbm, acc_sh.at[pl.ds(s * RPT, RPT)])
    plsc.subcore_barrier()
    _edge_pipeline(src_v, dst_v, u2_hbm, (rows0, rows1), acc_sh,
                   gsem, ssem, NCH, CH, idx_bufs=(idx0, idx1), shift=1,
                   q=c)
    plsc.subcore_barrier()
    pltpu.sync_copy(acc_sh.at[pl.ds(s * RPT, RPT)],
                    out_hbm.at[c, pl.ds(s * RPT, RPT)])


def _dinv_block(degp_ref):
    deg = degp_ref[0, :, 0] + degp_ref[1, :, 0] + 1.0
    return lax.rsqrt(jnp.maximum(deg, 1.0))


def _ux_body(x_ref, degp_ref, ux_ref):
    dinv = _dinv_block(degp_ref)
    ux_ref[...] = x_ref[...] * dinv[:, None]


def _l1_body(sx_ref, ux_ref, degp_ref, b1_ref, w1_ref, w2_ref, u2_ref,
             u2b_ref):
    dinv = _dinv_block(degp_ref)
    t = dinv[:, None] * (sx_ref[0] + sx_ref[1] + ux_ref[...])
    h1 = jnp.maximum(
        jnp.dot(t, w1_ref[...], preferred_element_type=_f32) + b1_ref[...],
        0.0)
    u2 = jnp.dot(h1, w2_ref[...],
                 preferred_element_type=_f32) * dinv[:, None]
    u2_ref[...] = u2
    u2b_ref[...] = u2.astype(_bf16)


def _mid_body(s_ref, u_ref, degp_ref, b_ref, w_ref, un_ref, unb_ref):
    dinv = _dinv_block(degp_ref)
    sc = jnp.concatenate([s_ref[0], s_ref[1]], axis=-1).astype(_f32)
    h = jnp.maximum(dinv[:, None] * (sc + u_ref[...]) + b_ref[...], 0.0)
    un = jnp.dot(h, w_ref[...],
                 preferred_element_type=_f32) * dinv[:, None]
    un_ref[...] = un
    unb_ref[...] = un.astype(_bf16)


def _final_body(s_ref, u_ref, degp_ref, b_ref, batch_ref,
                lw1_ref, lb1_ref, lw2_ref, lb2_ref, lw3_ref, lb3_ref,
                logp_ref, feat_ref, pool_acc, cnt_acc):
    i = pl.program_id(0)

    @pl.when(i == 0)
    def _():
        pool_acc[...] = jnp.zeros_like(pool_acc)
        cnt_acc[...] = jnp.zeros_like(cnt_acc)

    dinv = _dinv_block(degp_ref)
    sc = jnp.concatenate([s_ref[0], s_ref[1]], axis=-1).astype(_f32)
    h = jnp.maximum(dinv[:, None] * (sc + u_ref[...]) + b_ref[...], 0.0)
    b_row = batch_ref[0, 0, :]
    gi = lax.broadcasted_iota(_i32, (NG, RB), 0)
    onehot = (b_row[None, :] == gi).astype(_f32)
    pool_acc[...] += jnp.dot(onehot, h, preferred_element_type=_f32)
    cnt_acc[...] += jnp.sum(onehot, axis=1)[:, None]

    @pl.when(i == pl.num_programs(0) - 1)
    def _():
        counts = jnp.maximum(cnt_acc[:, 0:1], 1.0)
        g = pool_acc[...] / counts
        g = jnp.maximum(
            jnp.dot(g, lw1_ref[...], preferred_element_type=_f32)
            + lb1_ref[...], 0.0)
        feat = jnp.dot(g, lw2_ref[...],
                       preferred_element_type=_f32) + lb2_ref[...]
        logits = jnp.dot(feat, lw3_ref[...],
                         preferred_element_type=_f32) + lb3_ref[...]
        col = lax.broadcasted_iota(_i32, (NG, 128), 1)
        ml = jnp.where(col < 30, logits, jnp.full_like(logits, -1e30))
        mx = jnp.max(ml, axis=-1, keepdims=True)
        lse = mx + jnp.log(jnp.sum(jnp.exp(ml - mx), axis=-1, keepdims=True))
        logp_ref[...] = ml - lse
        feat_ref[...] = feat


def _row_spec(width):
    return pl.BlockSpec((RB, width), lambda i: (i, 0))


_S_SPEC = pl.BlockSpec((2, RB, HQ), lambda i: (0, i, 0))
_SX_SPEC = pl.BlockSpec((2, RB, XW), lambda i: (0, i, 0))
_DEG_SPEC = pl.BlockSpec((2, RB, XW), lambda i: (0, i, 0))


def _full_spec(shape):
    nd = len(shape)
    return pl.BlockSpec(shape, lambda i, _n=nd: (0,) * _n)


def kernel(x, edge_index, batch, W1, b1, W2, b2, W3, b3,
           lw1, lb1, lw2, lb2, lw3, lb3):
    src = edge_index[0].astype(_i32)
    dst = edge_index[1].astype(_i32)
    pad_e = EP - NE
    src_p = jnp.concatenate([src, jnp.zeros((pad_e,), _i32)])
    dst_p = jnp.concatenate([dst, jnp.full((pad_e,), NN, _i32)])
    src_w = src_p.reshape(2, 16, DROWS, 128)
    dst_w = dst_p.reshape(2, 16, DROWS, 128)
    src_msg = src_p.reshape(16, MROWS, 128)
    dst_msg = dst_p.reshape(16, MROWS, 128)
    batch3 = jnp.concatenate(
        [batch.astype(_i32),
         jnp.full((NP - NN,), NG, _i32)]).reshape(NRB, 1, RB)
    x16 = jnp.zeros((NP, XW), _f32).at[:NN, :7].set(x)
    W1p = jnp.zeros((XW, H), _f32).at[:7].set(W1)
    ones_deg = jnp.ones((128, XW), _f32)
    zeros_x = jnp.zeros((XZR, XW), _f32)
    zeros_msg = jnp.zeros((RPT, HQ), _bf16)
    lw3p = jnp.zeros((H, 128), _f32).at[:, :30].set(lw3)
    lb3p = jnp.zeros((1, 128), _f32).at[0, :30].set(lb3)
    b1r, b2r, b3r = (b.reshape(1, H) for b in (b1, b2, b3))
    lb1r, lb2r = lb1.reshape(1, H), lb2.reshape(1, 256)

    degp = _deg_kernel_fn()(dst_w, ones_deg, zeros_x)

    ux = pl.pallas_call(
        _ux_body,
        grid=(NRB,),
        in_specs=[_row_spec(XW), _DEG_SPEC],
        out_specs=_row_spec(XW),
        out_shape=jax.ShapeDtypeStruct((NP, XW), _f32),
    )(x16, degp)

    sx = _msgx_kernel_fn()(ux, src_w, dst_w, zeros_x)

    u2, u2b = pl.pallas_call(
        _l1_body,
        grid=(NRB,),
        in_specs=[_SX_SPEC, _row_spec(XW), _DEG_SPEC,
                  _full_spec((1, H)), _full_spec((XW, H)),
                  _full_spec((H, H))],
        out_specs=[_row_spec(H), _row_spec(H)],
        out_shape=[jax.ShapeDtypeStruct((NP, H), _f32),
                   jax.ShapeDtypeStruct((NP, H), _bf16)],
    )(sx, ux, degp, b1r, W1p, W2)

    def msg(u):
        return _msg_kernel_fn()(u.reshape(2 * NP, HQ), src_msg, dst_msg,
                                zeros_msg)

    s2 = msg(u2b)

    u3, u3b = pl.pallas_call(
        _mid_body,
        grid=(NRB,),
        in_specs=[_S_SPEC, _row_spec(H), _DEG_SPEC,
                  _full_spec((1, H)), _full_spec((H, H))],
        out_specs=[_row_spec(H), _row_spec(H)],
        out_shape=[jax.ShapeDtypeStruct((NP, H), _f32),
                   jax.ShapeDtypeStruct((NP, H), _bf16)],
    )(s2, u2, degp, b2r, W3)

    s3 = msg(u3b)

    logp_pad, feat = pl.pallas_call(
        _final_body,
        grid=(NRB,),
        in_specs=[_S_SPEC, _row_spec(H), _DEG_SPEC, _full_spec((1, H)),
                  pl.BlockSpec((1, 1, RB), lambda i: (i, 0, 0)),
                  _full_spec((H, H)), _full_spec((1, H)),
                  _full_spec((H, 256)), _full_spec((1, 256)),
                  _full_spec((H, 128)), _full_spec((1, 128))],
        out_specs=[pl.BlockSpec((NG, 128), lambda i: (0, 0)),
                   pl.BlockSpec((NG, 256), lambda i: (0, 0))],
        out_shape=[jax.ShapeDtypeStruct((NG, 128), _f32),
                   jax.ShapeDtypeStruct((NG, 256), _f32)],
        scratch_shapes=[pltpu.VMEM((NG, 256), _f32),
                        pltpu.VMEM((NG, 128), _f32)],
    )(s3, u3, degp, b3r, batch3, lw1, lb1r, lw2, lb2r, lw3p, lb3p)

    return (logp_pad[:, :30], feat)
